# baseline (speedup 1.0000x reference)
import jax
import jax.numpy as jnp
from jax.experimental import pallas as pl
from jax.experimental.pallas import tpu as pltpu
from jax.experimental.pallas import tpu_sc as plsc

D_IN = 2048
D_OUT = 2048
DEPTH = 4
N_NODES = 2 ** (DEPTH + 1) - 1
N_PAD = 128
TB = 1024


def _dot_nt(a, b):
    return jax.lax.dot_general(a, b, (((1,), (1,)), ((), ())),
                               preferred_element_type=jnp.float32)


def _logits_block(x_ref, w1_ref, l_ref):
    w1p = w1_ref[...]
    w1h = w1p.astype(jnp.bfloat16)
    rw = w1p - w1h.astype(jnp.float32)
    w1l = rw.astype(jnp.bfloat16)
    w1l2 = (rw - w1l.astype(jnp.float32)).astype(jnp.bfloat16)
    x = x_ref[...]
    xh = x.astype(jnp.bfloat16)
    r1 = x - xh.astype(jnp.float32)
    xl = r1.astype(jnp.bfloat16)
    xl2 = (r1 - xl.astype(jnp.float32)).astype(jnp.bfloat16)
    l_ref[...] = (_dot_nt(xh, w1h)
                  + (_dot_nt(xh, w1l) + _dot_nt(xl, w1h))
                  + (_dot_nt(xl, w1l) + _dot_nt(xl2, w1h) + _dot_nt(xh, w1l2)))


def _out_block(c_ref, w2_ref, o_ref):
    coef = c_ref[...]
    o_ref[...] = jax.lax.dot_general(
        coef.astype(jnp.bfloat16), w2_ref[...], (((1,), (0,)), ((), ())),
        preferred_element_type=jnp.float32)


_SC = plsc.get_sparse_core_info()
_NW = _SC.num_cores * _SC.num_subcores
_CHUNK_TOK = 256
_CHUNK_W = _CHUNK_TOK * N_PAD


def _route_kernel(l_ref, c_ref, lv, cv):
    per = l_ref.shape[0] // _NW
    wid = jax.lax.axis_index("s") * _SC.num_cores + jax.lax.axis_index("c")
    base = wid * per
    zero = jnp.zeros((16,), jnp.float32)
    lanes = jax.lax.iota(jnp.int32, 16)

    def zfull(i, c):
        cv[pl.ds(i * 16, 16)] = zero
        return c

    jax.lax.fori_loop(0, _CHUNK_W // 16, zfull, 0)

    def zrow(i, c):
        cv[pl.ds(i * N_PAD, 16)] = zero
        cv[pl.ds(i * N_PAD + 16, 16)] = zero
        return c

    def gbody(g, c):
        tok = g * 16 + lanes
        row = tok * N_PAD
        node = jnp.zeros((16,), jnp.int32)
        for _ in range(DEPTH + 1):
            idx = row + node
            l = plsc.load_gather(lv, [idx])
            plsc.store_scatter(cv, [idx], jnp.maximum(l, 0.0))
            node = 2 * node + 1 + (l > 0).astype(jnp.int32)
        return c

    for ch in range(per // _CHUNK_W):
        off = base + ch * _CHUNK_W
        pltpu.sync_copy(l_ref.at[pl.ds(off, _CHUNK_W)], lv)
        if ch:
            jax.lax.fori_loop(0, _CHUNK_TOK, zrow, 0)
        jax.lax.fori_loop(0, _CHUNK_TOK // 16, gbody, 0)
        pltpu.sync_copy(cv, c_ref.at[pl.ds(off, _CHUNK_W)])


def kernel(x, W1, W2):
    b = x.shape[0] * x.shape[1]
    xf = x.reshape(b, D_IN)
    w1p = jnp.pad(W1, ((0, N_PAD - N_NODES), (0, 0)))
    w2p = jnp.pad(W2, ((0, N_PAD - N_NODES), (0, 0))).astype(jnp.bfloat16)

    logits = pl.pallas_call(
        _logits_block,
        grid=(b // TB,),
        in_specs=[
            pl.BlockSpec((TB, D_IN), lambda i: (i, 0)),
            pl.BlockSpec((N_PAD, D_IN), lambda i: (0, 0)),
        ],
        out_specs=pl.BlockSpec((TB, N_PAD), lambda i: (i, 0)),
        out_shape=jax.ShapeDtypeStruct((b, N_PAD), jnp.float32),
    )(xf, w1p)

    mesh = plsc.VectorSubcoreMesh(core_axis_name="c", subcore_axis_name="s")
    coef = pl.kernel(
        _route_kernel,
        out_type=jax.ShapeDtypeStruct((b * N_PAD,), jnp.float32),
        mesh=mesh,
        compiler_params=pltpu.CompilerParams(needs_layout_passes=False),
        scratch_types=[
            pltpu.VMEM((_CHUNK_W,), jnp.float32),
            pltpu.VMEM((_CHUNK_W,), jnp.float32),
        ],
    )(logits.reshape(b * N_PAD))

    return pl.pallas_call(
        _out_block,
        grid=(b // TB,),
        in_specs=[
            pl.BlockSpec((TB, N_PAD), lambda i: (i, 0)),
            pl.BlockSpec((N_PAD, D_OUT), lambda i: (0, 0)),
        ],
        out_specs=pl.BlockSpec((TB, D_OUT), lambda i: (i, 0)),
        out_shape=jax.ShapeDtypeStruct((b, D_OUT), jnp.float32),
    )(coef.reshape(b, N_PAD), w2p)
